# baseline (speedup 1.0000x reference)
import functools

import jax
import jax.numpy as jnp
from jax.experimental import pallas as pl

_NEG_SLOPE = 0.01


def _lrelu(x):
    return jnp.where(x > 0, x, _NEG_SLOPE * x)


def _dot(a, b):
    return jax.lax.dot_general(a, b, (((1,), (0,)), ((), ())),
                               preferred_element_type=jnp.float32)


def _embed_body(ef_ref, sf_ref, w_ref, w1_ref, b1_ref, w2a_ref, w2b_ref,
                b2_ref, o_ref):
    ef = _dot(ef_ref[...], w1_ref[...]) + b1_ref[...]
    sf = (_dot(sf_ref[...], w2a_ref[...]) + w_ref[...] * w2b_ref[...]
          + b2_ref[...])
    o_ref[...] = jnp.concatenate([ef, sf], axis=1)


def _embed(elem_fea, sym_fea, elem_weights, p_elem, p_sym, block=2048):
    n, d_ef = elem_fea.shape
    d_sf = sym_fea.shape[1]
    w1, b1 = p_elem
    w2, b2 = p_sym
    w2a, w2b = w2[:d_sf], w2[d_sf:]
    f1, f2 = w1.shape[1], w2.shape[1]
    grid = (pl.cdiv(n, block),)
    return pl.pallas_call(
        _embed_body,
        grid=grid,
        in_specs=[
            pl.BlockSpec((block, d_ef), lambda i: (i, 0)),
            pl.BlockSpec((block, d_sf), lambda i: (i, 0)),
            pl.BlockSpec((block, 1), lambda i: (i, 0)),
            pl.BlockSpec(w1.shape, lambda i: (0, 0)),
            pl.BlockSpec((1, f1), lambda i: (0, 0)),
            pl.BlockSpec(w2a.shape, lambda i: (0, 0)),
            pl.BlockSpec((1, f2), lambda i: (0, 0)),
            pl.BlockSpec((1, f2), lambda i: (0, 0)),
        ],
        out_specs=pl.BlockSpec((block, f1 + f2), lambda i: (i, 0)),
        out_shape=jax.ShapeDtypeStruct((n, f1 + f2), jnp.float32),
    )(elem_fea, sym_fea, elem_weights, w1, b1.reshape(1, -1), w2a,
      w2b.reshape(1, -1), b2.reshape(1, -1))


def _pair_body(fs_ref, fn_ref, wg1a_ref, wg1b_ref, bg1_ref, wg2_ref, bg2_ref,
               wm1a_ref, wm1b_ref, bm1_ref, wm2_ref, bm2_ref,
               gate_ref, msg_ref):
    fs = fs_ref[...].astype(jnp.float32)
    fn = fn_ref[...].astype(jnp.float32)
    hg = _lrelu(_dot(fs, wg1a_ref[...]) + _dot(fn, wg1b_ref[...])
                + bg1_ref[...])
    gate_ref[...] = _dot(hg, wg2_ref[...]) + bg2_ref[...]
    hm = _lrelu(_dot(fs, wm1a_ref[...]) + _dot(fn, wm1b_ref[...])
                + bm1_ref[...])
    msg_ref[...] = _dot(hm, wm2_ref[...]) + bm2_ref[...]


def _pair_mlp(fs, fn, p, block=2048):
    n, d = fs.shape
    (wg1, bg1), = [p["gate"]["hidden"][0]]
    wg2, bg2 = p["gate"]["out"]
    (wm1, bm1), = [p["message"]["hidden"][0]]
    wm2, bm2 = p["message"]["out"]
    h = wg1.shape[1]
    dout = wm2.shape[1]
    grid = (pl.cdiv(n, block),)
    full = lambda w: pl.BlockSpec(w.shape, lambda i: (0, 0))
    row = lambda c: pl.BlockSpec((block, c), lambda i: (i, 0))
    return pl.pallas_call(
        _pair_body,
        grid=grid,
        in_specs=[row(d), row(d),
                  full(wg1[:d]), full(wg1[d:]), pl.BlockSpec((1, h), lambda i: (0, 0)),
                  full(wg2), pl.BlockSpec((1, 1), lambda i: (0, 0)),
                  full(wm1[:d]), full(wm1[d:]), pl.BlockSpec((1, h), lambda i: (0, 0)),
                  full(wm2), pl.BlockSpec((1, dout), lambda i: (0, 0))],
        out_specs=[row(1), row(dout)],
        out_shape=[jax.ShapeDtypeStruct((n, 1), jnp.float32),
                   jax.ShapeDtypeStruct((n, dout), jnp.float32)],
    )(fs, fn, wg1[:d], wg1[d:], bg1.reshape(1, -1), wg2, bg2.reshape(1, -1),
      wm1[:d], wm1[d:], bm1.reshape(1, -1), wm2, bm2.reshape(1, -1))


def _node_body(x_ref, wg1_ref, bg1_ref, wg2_ref, bg2_ref, wm1_ref, bm1_ref,
               wm2_ref, bm2_ref, gate_ref, msg_ref):
    x = x_ref[...]
    hg = _lrelu(_dot(x, wg1_ref[...]) + bg1_ref[...])
    gate_ref[...] = _dot(hg, wg2_ref[...]) + bg2_ref[...]
    hm = _lrelu(_dot(x, wm1_ref[...]) + bm1_ref[...])
    msg_ref[...] = _dot(hm, wm2_ref[...]) + bm2_ref[...]


def _node_mlp(x, p, block=2048):
    n, d = x.shape
    (wg1, bg1), = [p["gate"]["hidden"][0]]
    wg2, bg2 = p["gate"]["out"]
    (wm1, bm1), = [p["message"]["hidden"][0]]
    wm2, bm2 = p["message"]["out"]
    h = wg1.shape[1]
    dout = wm2.shape[1]
    grid = (pl.cdiv(n, block),)
    full = lambda w: pl.BlockSpec(w.shape, lambda i: (0, 0))
    row = lambda c: pl.BlockSpec((block, c), lambda i: (i, 0))
    return pl.pallas_call(
        _node_body,
        grid=grid,
        in_specs=[row(d),
                  full(wg1), pl.BlockSpec((1, h), lambda i: (0, 0)),
                  full(wg2), pl.BlockSpec((1, 1), lambda i: (0, 0)),
                  full(wm1), pl.BlockSpec((1, h), lambda i: (0, 0)),
                  full(wm2), pl.BlockSpec((1, dout), lambda i: (0, 0))],
        out_specs=[row(1), row(dout)],
        out_shape=[jax.ShapeDtypeStruct((n, 1), jnp.float32),
                   jax.ShapeDtypeStruct((n, dout), jnp.float32)],
    )(x, wg1, bg1.reshape(1, -1), wg2, bg2.reshape(1, -1),
      wm1, bm1.reshape(1, -1), wm2, bm2.reshape(1, -1))


def _segment_softmax_pool(gate, msg, index, weights, pw, num_segments):
    gmax = jax.ops.segment_max(gate, index, num_segments=num_segments,
                               indices_are_sorted=True)
    gate = (weights ** pw) * jnp.exp(gate - gmax[index])
    denom = jax.ops.segment_sum(gate, index, num_segments=num_segments,
                                indices_are_sorted=True)
    gate = gate / (denom[index] + 1e-10)
    return jax.ops.segment_sum(gate * msg, index, num_segments=num_segments,
                               indices_are_sorted=True)


@jax.jit
def _impl(elem_weights, elem_fea, sym_fea, self_fea_idx, nbr_fea_idx,
          cry_elem_idx, aug_cry_idx, params):
    n_cry = 12500
    n_aug = 6250
    fea = _embed(elem_fea, sym_fea, elem_weights,
                 params["elem_embed"], params["sym_embed"])
    n = fea.shape[0]
    for heads in params["graphs"]:
        nbr_w = elem_weights[nbr_fea_idx]
        fea_b = fea.astype(jnp.bfloat16)
        fs = fea_b[self_fea_idx]
        fn = fea_b[nbr_fea_idx]
        hf = []
        for h in heads:
            gate, msg = _pair_mlp(fs, fn, h)
            hf.append(_segment_softmax_pool(gate, msg, self_fea_idx, nbr_w,
                                            h["pow"], n))
        fea = jnp.mean(jnp.stack(hf), axis=0) + fea
    hf = []
    for h in params["cry_pool"]:
        gate, msg = _node_mlp(fea, h)
        hf.append(_segment_softmax_pool(gate, msg, cry_elem_idx, elem_weights,
                                        h["pow"], n_cry))
    cry = jnp.mean(jnp.stack(hf), axis=0)
    s = jax.ops.segment_sum(cry, aug_cry_idx, num_segments=n_aug,
                            indices_are_sorted=True)
    cnt = jax.ops.segment_sum(jnp.ones((cry.shape[0], 1), dtype=cry.dtype),
                              aug_cry_idx, num_segments=n_aug,
                              indices_are_sorted=True)
    return s / jnp.clip(cnt, 1.0, None)


def kernel(elem_weights, elem_fea, sym_fea, self_fea_idx, nbr_fea_idx,
           cry_elem_idx, aug_cry_idx, params):
    return _impl(elem_weights, elem_fea, sym_fea, self_fea_idx, nbr_fea_idx,
                 cry_elem_idx, aug_cry_idx, params)


# speedup vs baseline: 1.0079x; 1.0079x over previous
import functools

import jax
import jax.numpy as jnp
from jax.experimental import pallas as pl

_NEG_SLOPE = 0.01


def _lrelu(x):
    return jnp.where(x > 0, x, _NEG_SLOPE * x)


def _dot(a, b):
    return jax.lax.dot_general(a, b, (((1,), (0,)), ((), ())),
                               preferred_element_type=jnp.float32)


def _embed_body(ef_ref, sf_ref, w_ref, w1_ref, b1_ref, w2a_ref, w2b_ref,
                b2_ref, o_ref):
    ef = _dot(ef_ref[...], w1_ref[...]) + b1_ref[...]
    sf = (_dot(sf_ref[...], w2a_ref[...]) + w_ref[...] * w2b_ref[...]
          + b2_ref[...])
    o_ref[...] = jnp.concatenate([ef, sf], axis=1)


def _embed(elem_fea, sym_fea, elem_weights, p_elem, p_sym, block=2048):
    n, d_ef = elem_fea.shape
    d_sf = sym_fea.shape[1]
    w1, b1 = p_elem
    w2, b2 = p_sym
    w2a, w2b = w2[:d_sf], w2[d_sf:]
    f1, f2 = w1.shape[1], w2.shape[1]
    grid = (pl.cdiv(n, block),)
    return pl.pallas_call(
        _embed_body,
        grid=grid,
        in_specs=[
            pl.BlockSpec((block, d_ef), lambda i: (i, 0)),
            pl.BlockSpec((block, d_sf), lambda i: (i, 0)),
            pl.BlockSpec((block, 1), lambda i: (i, 0)),
            pl.BlockSpec(w1.shape, lambda i: (0, 0)),
            pl.BlockSpec((1, f1), lambda i: (0, 0)),
            pl.BlockSpec(w2a.shape, lambda i: (0, 0)),
            pl.BlockSpec((1, f2), lambda i: (0, 0)),
            pl.BlockSpec((1, f2), lambda i: (0, 0)),
        ],
        out_specs=pl.BlockSpec((block, f1 + f2), lambda i: (i, 0)),
        out_shape=jax.ShapeDtypeStruct((n, f1 + f2), jnp.float32),
    )(elem_fea, sym_fea, elem_weights, w1, b1.reshape(1, -1), w2a,
      w2b.reshape(1, -1), b2.reshape(1, -1))


def _pair_body(fs_ref, fn_ref, wg1a_ref, wg1b_ref, bg1_ref, wg2_ref, bg2_ref,
               wm1a_ref, wm1b_ref, bm1_ref, wm2_ref, bm2_ref,
               gate_ref, msg_ref):
    fs = fs_ref[...]
    fn = fn_ref[...]
    hg = _lrelu(_dot(fs, wg1a_ref[...]) + _dot(fn, wg1b_ref[...])
                + bg1_ref[...])
    gate_ref[...] = _dot(hg, wg2_ref[...]) + bg2_ref[...]
    hm = _lrelu(_dot(fs, wm1a_ref[...]) + _dot(fn, wm1b_ref[...])
                + bm1_ref[...])
    msg_ref[...] = _dot(hm, wm2_ref[...]) + bm2_ref[...]


def _pair_mlp(fs, fn, p, block=8192):
    n, d = fs.shape
    (wg1, bg1), = [p["gate"]["hidden"][0]]
    wg2, bg2 = p["gate"]["out"]
    (wm1, bm1), = [p["message"]["hidden"][0]]
    wm2, bm2 = p["message"]["out"]
    h = wg1.shape[1]
    dout = wm2.shape[1]
    grid = (pl.cdiv(n, block),)
    full = lambda w: pl.BlockSpec(w.shape, lambda i: (0, 0))
    row = lambda c: pl.BlockSpec((block, c), lambda i: (i, 0))
    return pl.pallas_call(
        _pair_body,
        grid=grid,
        in_specs=[row(d), row(d),
                  full(wg1[:d]), full(wg1[d:]), pl.BlockSpec((1, h), lambda i: (0, 0)),
                  full(wg2), pl.BlockSpec((1, 1), lambda i: (0, 0)),
                  full(wm1[:d]), full(wm1[d:]), pl.BlockSpec((1, h), lambda i: (0, 0)),
                  full(wm2), pl.BlockSpec((1, dout), lambda i: (0, 0))],
        out_specs=[row(1), row(dout)],
        out_shape=[jax.ShapeDtypeStruct((n, 1), jnp.float32),
                   jax.ShapeDtypeStruct((n, dout), jnp.float32)],
    )(fs, fn, wg1[:d], wg1[d:], bg1.reshape(1, -1), wg2, bg2.reshape(1, -1),
      wm1[:d], wm1[d:], bm1.reshape(1, -1), wm2, bm2.reshape(1, -1))


def _node_body(x_ref, wg1_ref, bg1_ref, wg2_ref, bg2_ref, wm1_ref, bm1_ref,
               wm2_ref, bm2_ref, gate_ref, msg_ref):
    x = x_ref[...]
    hg = _lrelu(_dot(x, wg1_ref[...]) + bg1_ref[...])
    gate_ref[...] = _dot(hg, wg2_ref[...]) + bg2_ref[...]
    hm = _lrelu(_dot(x, wm1_ref[...]) + bm1_ref[...])
    msg_ref[...] = _dot(hm, wm2_ref[...]) + bm2_ref[...]


def _node_mlp(x, p, block=2048):
    n, d = x.shape
    (wg1, bg1), = [p["gate"]["hidden"][0]]
    wg2, bg2 = p["gate"]["out"]
    (wm1, bm1), = [p["message"]["hidden"][0]]
    wm2, bm2 = p["message"]["out"]
    h = wg1.shape[1]
    dout = wm2.shape[1]
    grid = (pl.cdiv(n, block),)
    full = lambda w: pl.BlockSpec(w.shape, lambda i: (0, 0))
    row = lambda c: pl.BlockSpec((block, c), lambda i: (i, 0))
    return pl.pallas_call(
        _node_body,
        grid=grid,
        in_specs=[row(d),
                  full(wg1), pl.BlockSpec((1, h), lambda i: (0, 0)),
                  full(wg2), pl.BlockSpec((1, 1), lambda i: (0, 0)),
                  full(wm1), pl.BlockSpec((1, h), lambda i: (0, 0)),
                  full(wm2), pl.BlockSpec((1, dout), lambda i: (0, 0))],
        out_specs=[row(1), row(dout)],
        out_shape=[jax.ShapeDtypeStruct((n, 1), jnp.float32),
                   jax.ShapeDtypeStruct((n, dout), jnp.float32)],
    )(x, wg1, bg1.reshape(1, -1), wg2, bg2.reshape(1, -1),
      wm1, bm1.reshape(1, -1), wm2, bm2.reshape(1, -1))


def _segment_softmax_pool(gate, msg, index, weights, pw, num_segments):
    gmax = jax.ops.segment_max(gate, index, num_segments=num_segments,
                               indices_are_sorted=True)
    gate = (weights ** pw) * jnp.exp(gate - gmax[index])
    denom = jax.ops.segment_sum(gate, index, num_segments=num_segments,
                                indices_are_sorted=True)
    gate = gate / (denom[index] + 1e-10)
    return jax.ops.segment_sum(gate * msg, index, num_segments=num_segments,
                               indices_are_sorted=True)


@jax.jit
def _impl(elem_weights, elem_fea, sym_fea, self_fea_idx, nbr_fea_idx,
          cry_elem_idx, aug_cry_idx, params):
    n_cry = 12500
    n_aug = 6250
    fea = _embed(elem_fea, sym_fea, elem_weights,
                 params["elem_embed"], params["sym_embed"])
    n = fea.shape[0]
    for heads in params["graphs"]:
        nbr_w = elem_weights[nbr_fea_idx]
        fs = fea[self_fea_idx]
        fn = fea[nbr_fea_idx]
        hf = []
        for h in heads:
            gate, msg = _pair_mlp(fs, fn, h)
            hf.append(_segment_softmax_pool(gate, msg, self_fea_idx, nbr_w,
                                            h["pow"], n))
        fea = jnp.mean(jnp.stack(hf), axis=0) + fea
    hf = []
    for h in params["cry_pool"]:
        gate, msg = _node_mlp(fea, h)
        hf.append(_segment_softmax_pool(gate, msg, cry_elem_idx, elem_weights,
                                        h["pow"], n_cry))
    cry = jnp.mean(jnp.stack(hf), axis=0)
    s = jax.ops.segment_sum(cry, aug_cry_idx, num_segments=n_aug,
                            indices_are_sorted=True)
    cnt = jax.ops.segment_sum(jnp.ones((cry.shape[0], 1), dtype=cry.dtype),
                              aug_cry_idx, num_segments=n_aug,
                              indices_are_sorted=True)
    return s / jnp.clip(cnt, 1.0, None)


def kernel(elem_weights, elem_fea, sym_fea, self_fea_idx, nbr_fea_idx,
           cry_elem_idx, aug_cry_idx, params):
    return _impl(elem_weights, elem_fea, sym_fea, self_fea_idx, nbr_fea_idx,
                 cry_elem_idx, aug_cry_idx, params)
